# baseline (speedup 1.0000x reference)
import functools

import jax
import jax.numpy as jnp
from jax import lax
from jax.experimental import pallas as pl
from jax.experimental.pallas import tpu as pltpu

_V = 601
_C = 27
_B = 16384
_BLK = 16384
_STEPS = _B // _BLK


def _body(idx_ref, w_ref, out_ref, tab_ref):
    @pl.when(pl.program_id(0) == 0)
    def _():
        x = w_ref[...]
        e = jnp.exp(x)
        s = jnp.sum(e, axis=1, keepdims=True)
        tab_ref[...] = (e / s).astype(jnp.bfloat16)

    idx = idx_ref[...]
    rows = lax.broadcasted_iota(jnp.int32, (_BLK, _V), 1)
    onehot = (idx[:, None] == rows).astype(jnp.bfloat16)
    out_ref[...] = jnp.dot(onehot, tab_ref[...],
                           preferred_element_type=jnp.float32)


_lookup = pl.pallas_call(
    _body,
    grid=(_STEPS,),
    in_specs=[
        pl.BlockSpec((_BLK,), lambda i: (i,)),
        pl.BlockSpec((_V, _C), lambda i: (0, 0)),
    ],
    out_specs=pl.BlockSpec((_BLK, _C), lambda i: (i, 0)),
    out_shape=jax.ShapeDtypeStruct((_B, _C), jnp.float32),
    scratch_shapes=[pltpu.VMEM((_V, _C), jnp.bfloat16)],
    compiler_params=pltpu.CompilerParams(
        dimension_semantics=("arbitrary",)),
)


@jax.jit
def kernel(bigram_idx, W):
    return _lookup(bigram_idx.astype(jnp.int32), W)


# speedup vs baseline: 1.9378x; 1.9378x over previous
import jax
import jax.numpy as jnp
from jax.experimental import pallas as pl

_V = 601
_VP = 640
_C = 27
_B = 16384


def _body(idx_ref, wt_ref, out_ref):
    xt = wt_ref[...]
    et = jnp.exp(xt)
    st = jnp.sum(et, axis=0, keepdims=True)
    tab = et / st
    tab = jnp.concatenate(
        [tab, jnp.zeros((_C, _VP - _V), jnp.float32)], axis=1)

    idx = idx_ref[...]
    lo = jnp.broadcast_to((idx & 127)[None, :], (_C, _B))
    hi = jnp.broadcast_to((idx >> 7)[None, :], (_C, _B))
    out = jnp.zeros((_C, _B), jnp.float32)
    for b in range(_VP // 128):
        g = jnp.take_along_axis(tab[:, b * 128:(b + 1) * 128], lo, axis=1)
        out = jnp.where(hi == b, g, out)
    out_ref[...] = out


_lookup = pl.pallas_call(
    _body,
    out_shape=jax.ShapeDtypeStruct((_C, _B), jnp.float32),
)


@jax.jit
def kernel(bigram_idx, W):
    out_t = _lookup(bigram_idx.astype(jnp.int32), W.T)
    return out_t.T


# speedup vs baseline: 3.1048x; 1.6022x over previous
import jax
import jax.numpy as jnp
from jax import lax
from jax.experimental import pallas as pl

_V = 601
_VP = 640
_C = 27
_B = 16384


def _body(idx_ref, wt_ref, out_ref):
    xt = wt_ref[...]
    et = jnp.exp(xt)
    st = jnp.sum(et, axis=0, keepdims=True)
    tab = et / st
    tab = jnp.concatenate(
        [tab, jnp.zeros((32 - _C, _V), jnp.float32)], axis=0)
    tab = jnp.concatenate(
        [tab, jnp.zeros((32, _VP - _V), jnp.float32)], axis=1)

    tb = tab.astype(jnp.bfloat16)
    lo_u = lax.bitcast_convert_type(tb[:16], jnp.uint16).astype(jnp.uint32)
    hi_u = lax.bitcast_convert_type(tb[16:], jnp.uint16).astype(jnp.uint32)
    packed = (lo_u | (hi_u << 16)).astype(jnp.int32)

    idx = idx_ref[...]
    lo = jnp.broadcast_to((idx & 127)[None, :], (16, _B))
    hi = jnp.broadcast_to((idx >> 7)[None, :], (16, _B))
    out_p = jnp.zeros((16, _B), jnp.int32)
    for b in range(_VP // 128):
        g = jnp.take_along_axis(packed[:, b * 128:(b + 1) * 128], lo, axis=1)
        out_p = jnp.where(hi == b, g, out_p)

    up = lax.bitcast_convert_type(out_p, jnp.uint32)
    low_f = lax.bitcast_convert_type(
        (up & 0xFFFF).astype(jnp.uint16), jnp.bfloat16).astype(jnp.float32)
    high_f = lax.bitcast_convert_type(
        (up >> 16).astype(jnp.uint16), jnp.bfloat16).astype(jnp.float32)
    out_ref[...] = jnp.concatenate([low_f, high_f[: _C - 16]], axis=0)


_lookup = pl.pallas_call(
    _body,
    out_shape=jax.ShapeDtypeStruct((_C, _B), jnp.float32),
)


@jax.jit
def kernel(bigram_idx, W):
    out_t = _lookup(bigram_idx.astype(jnp.int32), W.T)
    return out_t.T


# speedup vs baseline: 3.2249x; 1.0387x over previous
import jax
import jax.numpy as jnp
from jax import lax
from jax.experimental import pallas as pl
from jax.experimental.pallas import tpu as pltpu

_V = 601
_VP = 640
_C = 27
_B = 16384
_BLK = 4096
_STEPS = _B // _BLK


def _body(idx_ref, wt_ref, out_ref, tab_ref):
    @pl.when(pl.program_id(0) == 0)
    def _():
        xt = wt_ref[...]
        et = jnp.exp(xt)
        st = jnp.sum(et, axis=0, keepdims=True)
        tab = et / st
        tab = jnp.concatenate(
            [tab, jnp.zeros((32 - _C, _V), jnp.float32)], axis=0)
        tab = jnp.concatenate(
            [tab, jnp.zeros((32, _VP - _V), jnp.float32)], axis=1)
        tb = tab.astype(jnp.bfloat16)
        lo_u = lax.bitcast_convert_type(tb[:16], jnp.uint16).astype(
            jnp.uint32)
        hi_u = lax.bitcast_convert_type(tb[16:], jnp.uint16).astype(
            jnp.uint32)
        tab_ref[...] = (lo_u | (hi_u << 16)).astype(jnp.int32)

    packed = tab_ref[...]
    idx = idx_ref[...]
    lo = jnp.broadcast_to((idx & 127)[None, :], (16, _BLK))
    hi = jnp.broadcast_to((idx >> 7)[None, :], (16, _BLK))
    out_p = jnp.zeros((16, _BLK), jnp.int32)
    for b in range(_VP // 128):
        g = jnp.take_along_axis(packed[:, b * 128:(b + 1) * 128], lo, axis=1)
        out_p = jnp.where(hi == b, g, out_p)

    up = lax.bitcast_convert_type(out_p, jnp.uint32)
    low_f = lax.bitcast_convert_type(
        (up & 0xFFFF).astype(jnp.uint16), jnp.bfloat16).astype(jnp.float32)
    high_f = lax.bitcast_convert_type(
        (up >> 16).astype(jnp.uint16), jnp.bfloat16).astype(jnp.float32)
    out_ref[...] = jnp.concatenate([low_f, high_f[: _C - 16]], axis=0)


_lookup = pl.pallas_call(
    _body,
    grid=(_STEPS,),
    in_specs=[
        pl.BlockSpec((_BLK,), lambda i: (i,)),
        pl.BlockSpec((_C, _V), lambda i: (0, 0)),
    ],
    out_specs=pl.BlockSpec((_C, _BLK), lambda i: (0, i)),
    out_shape=jax.ShapeDtypeStruct((_C, _B), jnp.float32),
    scratch_shapes=[pltpu.VMEM((16, _VP), jnp.int32)],
    compiler_params=pltpu.CompilerParams(
        dimension_semantics=("arbitrary",)),
)


@jax.jit
def kernel(bigram_idx, W):
    out_t = _lookup(bigram_idx.astype(jnp.int32), W.T)
    return out_t.T
